# baseline (speedup 1.0000x reference)
import functools

import jax
import jax.numpy as jnp
from jax.experimental import pallas as pl
from jax.experimental.pallas import tpu as pltpu

B = 8
N = 4096
N1 = 1024
N2 = 256
DIN = 3
NC = 20

_EPS_D = 1e-8
_EPS_BN = 1e-5
_BIG_F = 1e9


def _split_bf16(x):
    hi = x.astype(jnp.bfloat16)
    lo = (x - hi.astype(jnp.float32)).astype(jnp.bfloat16)
    return hi, lo


def _dot3(a, b):
    ahi, alo = _split_bf16(a)
    bhi, blo = _split_bf16(b)
    d = lambda x, y: jnp.dot(x, y, preferred_element_type=jnp.float32)
    return d(ahi, bhi) + (d(ahi, blo) + d(alo, bhi))


def _top3_select(d):
    iota0 = jax.lax.broadcasted_iota(jnp.int32, d.shape, 0).astype(jnp.float32)
    mins = []
    preds = []
    for k in range(3):
        mval = jnp.min(d, axis=0, keepdims=True)
        idxk = jnp.min(jnp.where(d == mval, iota0, _BIG_F), axis=0,
                       keepdims=True)
        pred = iota0 == idxk
        mins.append(mval)
        preds.append(pred)
        if k < 2:
            d = jnp.where(pred, jnp.inf, d)
    return mins, preds


def _top3_weights(d, m):
    mins, preds = _top3_select(d)
    recips = [1.0 / (dk + _EPS_D) for dk in mins]
    norm = recips[0] + recips[1] + recips[2]
    wt = jnp.zeros(d.shape, jnp.float32)
    for rk, pred in zip(recips, preds):
        wt = jnp.where(pred, rk / norm, wt)
    return wt


def _top3_weights_bf16(d):
    return _top3_weights(d, d.shape[0]).astype(jnp.bfloat16)


def _sqdist(kxyz, uxyz_c, m, n):
    d = jnp.zeros((m, n), jnp.float32)
    for c in range(3):
        diff = kxyz[:, c:c + 1] - uxyz_c[c:c + 1, :]
        d = d + diff * diff
    return d


def _accum_sums(s_ref, y, first):
    part = jnp.concatenate(
        [jnp.sum(y, axis=1, keepdims=True),
         jnp.sum(y * y, axis=1, keepdims=True)], axis=1)

    @pl.when(first)
    def _():
        s_ref[...] = jnp.zeros_like(s_ref)

    s_ref[...] += part


def _sums_of(y):
    return jnp.concatenate(
        [jnp.sum(y, axis=1, keepdims=True),
         jnp.sum(y * y, axis=1, keepdims=True)], axis=1)


def _fp1_chain_body(uxyz_ref, kxyz_ref, w1a_ref, bb_ref, f_ref, w1b_ref,
                    b1_ref, g1_ref, be1_ref, w2_ref, b2_ref, g2_ref, be2_ref,
                    w1a2_ref, ghi_ref, glo_ref, y_scr):
    s1 = jnp.zeros((256, 2), jnp.float32)
    for b in range(B):
        d = _sqdist(kxyz_ref[b], uxyz_ref[b], N2, N1)
        wt = _top3_weights(d, N2)
        hfeat = _dot3(w1a_ref[...], bb_ref[b])
        y = _dot3(hfeat, wt) + _dot3(w1b_ref[...], f_ref[b]) + b1_ref[...]
        y_scr[b] = y
        s1 = s1 + _sums_of(y)
    scale1, shift1 = _bn_scale_shift_v(s1, g1_ref[...], be1_ref[...],
                                       float(B * N1))
    s2 = jnp.zeros((256, 2), jnp.float32)
    for b in range(B):
        a = jnp.maximum(y_scr[b] * scale1 + shift1, 0.0)
        z = _dot3(w2_ref[...], a) + b2_ref[...]
        y_scr[b] = z
        s2 = s2 + _sums_of(z)
    scale2, shift2 = _bn_scale_shift_v(s2, g2_ref[...], be2_ref[...],
                                       float(B * N1))
    for b in range(B):
        a = jnp.maximum(y_scr[b] * scale2 + shift2, 0.0)
        g = _dot3(w1a2_ref[...], a)
        ghi, glo = _split_bf16(g)
        ghi_ref[b] = ghi
        glo_ref[b] = glo


def _bn_scale_shift_v(s, g, be, count):
    mean = s[:, 0:1] / count
    var = s[:, 1:2] / count - mean * mean
    inv = jax.lax.rsqrt(var + _EPS_BN)
    scale = g * inv
    shift = be - mean * scale
    return scale, shift


def _bn_scale_shift(s_ref, g_ref, be_ref, count):
    s = s_ref[...]
    mean = s[:, 0:1] / count
    var = s[:, 1:2] / count - mean * mean
    inv = jax.lax.rsqrt(var + _EPS_BN)
    scale = g_ref[...] * inv
    shift = be_ref[...] - mean * scale
    return scale, shift


def _bn_relu_conv_body(count, x_ref, s_in_ref, g_ref, be_ref, w_ref, b_ref,
                       z_ref, s_out_ref):
    first = pl.program_id(0) == 0
    scale, shift = _bn_scale_shift(s_in_ref, g_ref, be_ref, count)
    a = jnp.maximum(x_ref[0] * scale + shift, 0.0)
    z = _dot3(w_ref[...], a)
    if b_ref is not None:
        z = z + b_ref[...]
    z_ref[0] = z
    if s_out_ref is not None:
        _accum_sums(s_out_ref, z, first)


def _cls_from_z_body(count, z_ref, s4_ref, g2_ref, be2_ref, s5_ref, g3_ref,
                     be3_ref, w_ref, b_ref, out_ref):
    scale2, shift2 = _bn_scale_shift(s4_ref, g2_ref, be2_ref, count)
    scale3, shift3 = _bn_scale_shift(s5_ref, g3_ref, be3_ref, count)
    f = jnp.maximum(z_ref[0] * scale2 + shift2, 0.0)
    a = jnp.maximum(f * scale3 + shift3, 0.0)
    out_ref[0] = _dot3(w_ref[...], a) + b_ref[...]


def _bn_relu_conv_split_body(count, x_ref, s_in_ref, g_ref, be_ref, w_ref,
                             zhi_ref, zlo_ref):
    scale, shift = _bn_scale_shift(s_in_ref, g_ref, be_ref, count)
    a = jnp.maximum(x_ref[0] * scale + shift, 0.0)
    z = _dot3(w_ref[...], a)
    zhi, zlo = _split_bf16(z)
    zhi_ref[0] = zhi
    zlo_ref[0] = zlo


def _bn_relu_body(count, x_ref, s_in_ref, g_ref, be_ref, f_ref, s_out_ref):
    first = pl.program_id(0) == 0
    scale, shift = _bn_scale_shift(s_in_ref, g_ref, be_ref, count)
    f = jnp.maximum(x_ref[0] * scale + shift, 0.0)
    f_ref[0] = f
    _accum_sums(s_out_ref, f, first)


def _fp2_body(uxyz_ref, kxyz_ref, xf_ref, g1hi_ref, g1lo_ref, w1b_ref, b1_ref,
              y_ref, s_ref):
    b = pl.program_id(0)
    i = pl.program_id(1)
    nt = y_ref.shape[2]
    d = _sqdist(kxyz_ref[0], uxyz_ref[0], N1, nt)
    wt32 = _top3_weights(d, N1)
    wt_hi, wt_lo = _split_bf16(wt32)
    pf32 = jnp.float32
    g1hi = g1hi_ref[0]
    y = (jnp.dot(g1hi, wt_hi, preferred_element_type=pf32)
         + jnp.dot(g1lo_ref[0], wt_hi, preferred_element_type=pf32)
         + jnp.dot(g1hi, wt_lo, preferred_element_type=pf32))
    xf = xf_ref[0]
    w1b = w1b_ref[...]
    for c in range(DIN):
        y = y + w1b[:, c:c + 1] * xf[c:c + 1, :]
    y = y + b1_ref[...]
    y_ref[0] = y
    _accum_sums(s_ref, y, jnp.logical_and(b == 0, i == 0))


def _col(v):
    return v.reshape(-1, 1)


def kernel(input_xyz, sa1_xyz, sa2_xyz, input_features, sa1_features,
           backbone_feat, fp1_w1, fp1_b1, fp1_g1, fp1_be1, fp1_w2, fp1_b2,
           fp1_g2, fp1_be2, fp2_w1, fp2_b1, fp2_g1, fp2_be1, fp2_w2, fp2_b2,
           fp2_g2, fp2_be2, cls_g, cls_be, cls_w, cls_b):
    f32 = jnp.float32
    sa1_xyz_c = sa1_xyz.transpose(0, 2, 1)
    input_xyz_c = input_xyz.transpose(0, 2, 1)
    w1a_fp1 = fp1_w1[:, :256]
    w1b_fp1 = fp1_w1[:, 256:]
    w1a_fp2 = fp2_w1[:, :256]
    w1b_fp2 = fp2_w1[:, 256:]

    full = lambda shp: pl.BlockSpec(shp, lambda b: tuple(0 for _ in shp))
    perb = lambda shp: pl.BlockSpec(
        (1,) + shp, lambda b: (b,) + tuple(0 for _ in shp))

    M1 = float(B * N1)
    M2 = float(B * N)

    whole = lambda shp: pl.BlockSpec(shp, lambda: tuple(0 for _ in shp))
    g1hi, g1lo = pl.pallas_call(
        _fp1_chain_body,
        grid=(),
        in_specs=[whole((B, 3, N1)), whole((B, N2, 3)), whole((256, 256)),
                  whole((B, 256, N2)), whole((B, 128, N1)),
                  whole((256, 128)), whole((256, 1)), whole((256, 1)),
                  whole((256, 1)), whole((256, 256)), whole((256, 1)),
                  whole((256, 1)), whole((256, 1)), whole((256, 256))],
        out_specs=[whole((B, 256, N1)), whole((B, 256, N1))],
        out_shape=[jax.ShapeDtypeStruct((B, 256, N1), jnp.bfloat16),
                   jax.ShapeDtypeStruct((B, 256, N1), jnp.bfloat16)],
        scratch_shapes=[pltpu.VMEM((B, 256, N1), f32)],
    )(sa1_xyz_c, sa2_xyz, w1a_fp1, backbone_feat, sa1_features, w1b_fp1,
      _col(fp1_b1), _col(fp1_g1), _col(fp1_be1), fp1_w2, _col(fp1_b2),
      _col(fp1_g2), _col(fp1_be2), w1a_fp2)

    NT = 512
    nsteps = N // NT
    y2, s3 = pl.pallas_call(
        _fp2_body,
        grid=(B, nsteps),
        in_specs=[
            pl.BlockSpec((1, 3, NT), lambda b, i: (b, 0, i)),
            pl.BlockSpec((1, N1, 3), lambda b, i: (b, 0, 0)),
            pl.BlockSpec((1, DIN, NT), lambda b, i: (b, 0, i)),
            pl.BlockSpec((1, 256, N1), lambda b, i: (b, 0, 0)),
            pl.BlockSpec((1, 256, N1), lambda b, i: (b, 0, 0)),
            pl.BlockSpec((256, DIN), lambda b, i: (0, 0)),
            pl.BlockSpec((256, 1), lambda b, i: (0, 0)),
        ],
        out_specs=[pl.BlockSpec((1, 256, NT), lambda b, i: (b, 0, i)),
                   pl.BlockSpec((256, 2), lambda b, i: (0, 0))],
        out_shape=[jax.ShapeDtypeStruct((B, 256, N), f32),
                   jax.ShapeDtypeStruct((256, 2), f32)],
    )(input_xyz_c, sa1_xyz, input_features, g1hi, g1lo, w1b_fp2,
      _col(fp2_b1))

    z2, s4 = pl.pallas_call(
        functools.partial(_bn_relu_conv_body, M2),
        grid=(B,),
        in_specs=[perb((256, N)), full((256, 2)), full((256, 1)),
                  full((256, 1)), full((256, 256)), full((256, 1))],
        out_specs=[perb((256, N)), full((256, 2))],
        out_shape=[jax.ShapeDtypeStruct((B, 256, N), f32),
                   jax.ShapeDtypeStruct((256, 2), f32)],
    )(y2, s3, _col(fp2_g1), _col(fp2_be1), fp2_w2, _col(fp2_b2))

    f2, s5 = pl.pallas_call(
        functools.partial(_bn_relu_body, M2),
        grid=(B,),
        in_specs=[perb((256, N)), full((256, 2)), full((256, 1)),
                  full((256, 1))],
        out_specs=[perb((256, N)), full((256, 2))],
        out_shape=[jax.ShapeDtypeStruct((B, 256, N), f32),
                   jax.ShapeDtypeStruct((256, 2), f32)],
    )(z2, s4, _col(fp2_g2), _col(fp2_be2))

    pred = pl.pallas_call(
        functools.partial(_cls_from_z_body, M2),
        grid=(B,),
        in_specs=[perb((256, N)), full((256, 2)), full((256, 1)),
                  full((256, 1)), full((256, 2)), full((256, 1)),
                  full((256, 1)), full((NC, 256)), full((NC, 1))],
        out_specs=perb((NC, N)),
        out_shape=jax.ShapeDtypeStruct((B, NC, N), f32),
    )(z2, s4, _col(fp2_g2), _col(fp2_be2), s5, _col(cls_g), _col(cls_be),
      cls_w, _col(cls_b))

    return (f2, pred)


# speedup vs baseline: 1.0321x; 1.0321x over previous
import functools

import jax
import jax.numpy as jnp
from jax.experimental import pallas as pl
from jax.experimental.pallas import tpu as pltpu

B = 8
N = 4096
N1 = 1024
N2 = 256
DIN = 3
NC = 20

_EPS_D = 1e-8
_EPS_BN = 1e-5
_BIG_F = 1e9


def _split_bf16(x):
    hi = x.astype(jnp.bfloat16)
    lo = (x - hi.astype(jnp.float32)).astype(jnp.bfloat16)
    return hi, lo


def _dot3(a, b):
    ahi, alo = _split_bf16(a)
    bhi, blo = _split_bf16(b)
    d = lambda x, y: jnp.dot(x, y, preferred_element_type=jnp.float32)
    return d(ahi, bhi) + (d(ahi, blo) + d(alo, bhi))


def _top3_select(d):
    iota0 = jax.lax.broadcasted_iota(jnp.int32, d.shape, 0).astype(jnp.float32)
    mins = []
    preds = []
    for k in range(3):
        mval = jnp.min(d, axis=0, keepdims=True)
        idxk = jnp.min(jnp.where(d == mval, iota0, _BIG_F), axis=0,
                       keepdims=True)
        pred = iota0 == idxk
        mins.append(mval)
        preds.append(pred)
        if k < 2:
            d = jnp.where(pred, jnp.inf, d)
    return mins, preds


def _top3_weights(d, m):
    mins, preds = _top3_select(d)
    recips = [1.0 / (dk + _EPS_D) for dk in mins]
    norm = recips[0] + recips[1] + recips[2]
    wt = jnp.zeros(d.shape, jnp.float32)
    for rk, pred in zip(recips, preds):
        wt = jnp.where(pred, rk / norm, wt)
    return wt


def _sqdist(kxyz, uxyz_c, m, n):
    d = jnp.zeros((m, n), jnp.float32)
    for c in range(3):
        diff = kxyz[:, c:c + 1] - uxyz_c[c:c + 1, :]
        d = d + diff * diff
    return d


def _accum_sums(s_ref, y, first):
    part = jnp.concatenate(
        [jnp.sum(y, axis=1, keepdims=True),
         jnp.sum(y * y, axis=1, keepdims=True)], axis=1)

    @pl.when(first)
    def _():
        s_ref[...] = jnp.zeros_like(s_ref)

    s_ref[...] += part


def _sums_of(y):
    return jnp.concatenate(
        [jnp.sum(y, axis=1, keepdims=True),
         jnp.sum(y * y, axis=1, keepdims=True)], axis=1)


def _fp1_chain_body(uxyz_ref, kxyz_ref, w1a_ref, bb_ref, f_ref, w1b_ref,
                    b1_ref, g1_ref, be1_ref, w2_ref, b2_ref, g2_ref, be2_ref,
                    w1a2_ref, ghi_ref, glo_ref, y_scr):
    s1 = jnp.zeros((256, 2), jnp.float32)
    for b in range(B):
        d = _sqdist(kxyz_ref[b], uxyz_ref[b], N2, N1)
        wt = _top3_weights(d, N2)
        hfeat = _dot3(w1a_ref[...], bb_ref[b])
        y = _dot3(hfeat, wt) + _dot3(w1b_ref[...], f_ref[b]) + b1_ref[...]
        y_scr[b] = y
        s1 = s1 + _sums_of(y)
    scale1, shift1 = _bn_scale_shift_v(s1, g1_ref[...], be1_ref[...],
                                       float(B * N1))
    s2 = jnp.zeros((256, 2), jnp.float32)
    for b in range(B):
        a = jnp.maximum(y_scr[b] * scale1 + shift1, 0.0)
        z = _dot3(w2_ref[...], a) + b2_ref[...]
        y_scr[b] = z
        s2 = s2 + _sums_of(z)
    scale2, shift2 = _bn_scale_shift_v(s2, g2_ref[...], be2_ref[...],
                                       float(B * N1))
    for b in range(B):
        a = jnp.maximum(y_scr[b] * scale2 + shift2, 0.0)
        g = _dot3(w1a2_ref[...], a)
        ghi, glo = _split_bf16(g)
        ghi_ref[b] = ghi
        glo_ref[b] = glo


def _bn_scale_shift_v(s, g, be, count):
    mean = s[:, 0:1] / count
    var = s[:, 1:2] / count - mean * mean
    inv = jax.lax.rsqrt(var + _EPS_BN)
    scale = g * inv
    shift = be - mean * scale
    return scale, shift


def _bn_scale_shift(s_ref, g_ref, be_ref, count):
    s = s_ref[...]
    mean = s[:, 0:1] / count
    var = s[:, 1:2] / count - mean * mean
    inv = jax.lax.rsqrt(var + _EPS_BN)
    scale = g_ref[...] * inv
    shift = be_ref[...] - mean * scale
    return scale, shift


def _bn_relu_conv_body(count, x_ref, s_in_ref, g_ref, be_ref, w_ref, b_ref,
                       z_ref, s_out_ref):
    first = pl.program_id(0) == 0
    scale, shift = _bn_scale_shift(s_in_ref, g_ref, be_ref, count)
    a = jnp.maximum(x_ref[0] * scale + shift, 0.0)
    z = _dot3(w_ref[...], a)
    if b_ref is not None:
        z = z + b_ref[...]
    z_ref[0] = z
    if s_out_ref is not None:
        _accum_sums(s_out_ref, z, first)


def _cls_from_z_body(count, z_ref, s4_ref, g2_ref, be2_ref, s5_ref, g3_ref,
                     be3_ref, w_ref, b_ref, out_ref):
    scale2, shift2 = _bn_scale_shift(s4_ref, g2_ref, be2_ref, count)
    scale3, shift3 = _bn_scale_shift(s5_ref, g3_ref, be3_ref, count)
    f = jnp.maximum(z_ref[0] * scale2 + shift2, 0.0)
    a = jnp.maximum(f * scale3 + shift3, 0.0)
    out_ref[0] = _dot3(w_ref[...], a) + b_ref[...]


def _bn_relu_body(count, x_ref, s_in_ref, g_ref, be_ref, f_ref, s_out_ref):
    first = pl.program_id(0) == 0
    scale, shift = _bn_scale_shift(s_in_ref, g_ref, be_ref, count)
    f = jnp.maximum(x_ref[0] * scale + shift, 0.0)
    f_ref[0] = f
    _accum_sums(s_out_ref, f, first)


def _fp2_body(uxyz_ref, kxyz_ref, xf_ref, g1hi_ref, g1lo_ref, w1b_ref, b1_ref,
              y_ref, s_ref):
    b = pl.program_id(0)
    i = pl.program_id(1)
    nt = y_ref.shape[2]
    d = _sqdist(kxyz_ref[0], uxyz_ref[0], N1, nt)
    wt32 = _top3_weights(d, N1)
    wt_hi, wt_lo = _split_bf16(wt32)
    pf32 = jnp.float32
    g1hi = g1hi_ref[0]
    y = (jnp.dot(g1hi, wt_hi, preferred_element_type=pf32)
         + jnp.dot(g1lo_ref[0], wt_hi, preferred_element_type=pf32)
         + jnp.dot(g1hi, wt_lo, preferred_element_type=pf32))
    xf = xf_ref[0]
    w1b = w1b_ref[...]
    for c in range(DIN):
        y = y + w1b[:, c:c + 1] * xf[c:c + 1, :]
    y = y + b1_ref[...]
    y_ref[0] = y
    _accum_sums(s_ref, y, jnp.logical_and(b == 0, i == 0))


def _col(v):
    return v.reshape(-1, 1)


def kernel(input_xyz, sa1_xyz, sa2_xyz, input_features, sa1_features,
           backbone_feat, fp1_w1, fp1_b1, fp1_g1, fp1_be1, fp1_w2, fp1_b2,
           fp1_g2, fp1_be2, fp2_w1, fp2_b1, fp2_g1, fp2_be1, fp2_w2, fp2_b2,
           fp2_g2, fp2_be2, cls_g, cls_be, cls_w, cls_b):
    f32 = jnp.float32
    sa1_xyz_c = sa1_xyz.transpose(0, 2, 1)
    input_xyz_c = input_xyz.transpose(0, 2, 1)
    w1a_fp1 = fp1_w1[:, :256]
    w1b_fp1 = fp1_w1[:, 256:]
    w1a_fp2 = fp2_w1[:, :256]
    w1b_fp2 = fp2_w1[:, 256:]

    full = lambda shp: pl.BlockSpec(shp, lambda b: tuple(0 for _ in shp))
    perb = lambda shp: pl.BlockSpec(
        (1,) + shp, lambda b: (b,) + tuple(0 for _ in shp))

    M1 = float(B * N1)
    M2 = float(B * N)

    whole = lambda shp: pl.BlockSpec(shp, lambda: tuple(0 for _ in shp))
    g1hi, g1lo = pl.pallas_call(
        _fp1_chain_body,
        grid=(),
        in_specs=[whole((B, 3, N1)), whole((B, N2, 3)), whole((256, 256)),
                  whole((B, 256, N2)), whole((B, 128, N1)),
                  whole((256, 128)), whole((256, 1)), whole((256, 1)),
                  whole((256, 1)), whole((256, 256)), whole((256, 1)),
                  whole((256, 1)), whole((256, 1)), whole((256, 256))],
        out_specs=[whole((B, 256, N1)), whole((B, 256, N1))],
        out_shape=[jax.ShapeDtypeStruct((B, 256, N1), jnp.bfloat16),
                   jax.ShapeDtypeStruct((B, 256, N1), jnp.bfloat16)],
        scratch_shapes=[pltpu.VMEM((B, 256, N1), f32)],
    )(sa1_xyz_c, sa2_xyz, w1a_fp1, backbone_feat, sa1_features, w1b_fp1,
      _col(fp1_b1), _col(fp1_g1), _col(fp1_be1), fp1_w2, _col(fp1_b2),
      _col(fp1_g2), _col(fp1_be2), w1a_fp2)

    NT = 1024
    nsteps = N // NT
    y2, s3 = pl.pallas_call(
        _fp2_body,
        grid=(B, nsteps),
        in_specs=[
            pl.BlockSpec((1, 3, NT), lambda b, i: (b, 0, i)),
            pl.BlockSpec((1, N1, 3), lambda b, i: (b, 0, 0)),
            pl.BlockSpec((1, DIN, NT), lambda b, i: (b, 0, i)),
            pl.BlockSpec((1, 256, N1), lambda b, i: (b, 0, 0)),
            pl.BlockSpec((1, 256, N1), lambda b, i: (b, 0, 0)),
            pl.BlockSpec((256, DIN), lambda b, i: (0, 0)),
            pl.BlockSpec((256, 1), lambda b, i: (0, 0)),
        ],
        out_specs=[pl.BlockSpec((1, 256, NT), lambda b, i: (b, 0, i)),
                   pl.BlockSpec((256, 2), lambda b, i: (0, 0))],
        out_shape=[jax.ShapeDtypeStruct((B, 256, N), f32),
                   jax.ShapeDtypeStruct((256, 2), f32)],
    )(input_xyz_c, sa1_xyz, input_features, g1hi, g1lo, w1b_fp2,
      _col(fp2_b1))

    z2, s4 = pl.pallas_call(
        functools.partial(_bn_relu_conv_body, M2),
        grid=(B,),
        in_specs=[perb((256, N)), full((256, 2)), full((256, 1)),
                  full((256, 1)), full((256, 256)), full((256, 1))],
        out_specs=[perb((256, N)), full((256, 2))],
        out_shape=[jax.ShapeDtypeStruct((B, 256, N), f32),
                   jax.ShapeDtypeStruct((256, 2), f32)],
    )(y2, s3, _col(fp2_g1), _col(fp2_be1), fp2_w2, _col(fp2_b2))

    f2, s5 = pl.pallas_call(
        functools.partial(_bn_relu_body, M2),
        grid=(B,),
        in_specs=[perb((256, N)), full((256, 2)), full((256, 1)),
                  full((256, 1))],
        out_specs=[perb((256, N)), full((256, 2))],
        out_shape=[jax.ShapeDtypeStruct((B, 256, N), f32),
                   jax.ShapeDtypeStruct((256, 2), f32)],
    )(z2, s4, _col(fp2_g2), _col(fp2_be2))

    pred = pl.pallas_call(
        functools.partial(_cls_from_z_body, M2),
        grid=(B,),
        in_specs=[perb((256, N)), full((256, 2)), full((256, 1)),
                  full((256, 1)), full((256, 2)), full((256, 1)),
                  full((256, 1)), full((NC, 256)), full((NC, 1))],
        out_specs=perb((NC, N)),
        out_shape=jax.ShapeDtypeStruct((B, NC, N), f32),
    )(z2, s4, _col(fp2_g2), _col(fp2_be2), s5, _col(cls_g), _col(cls_be),
      cls_w, _col(cls_b))

    return (f2, pred)


# speedup vs baseline: 1.0329x; 1.0008x over previous
import functools

import jax
import jax.numpy as jnp
from jax.experimental import pallas as pl
from jax.experimental.pallas import tpu as pltpu

B = 8
N = 4096
N1 = 1024
N2 = 256
DIN = 3
NC = 20

_EPS_D = 1e-8
_EPS_BN = 1e-5
_BIG_F = 1e9


def _split_bf16(x):
    hi = x.astype(jnp.bfloat16)
    lo = (x - hi.astype(jnp.float32)).astype(jnp.bfloat16)
    return hi, lo


def _dot3(a, b):
    ahi, alo = _split_bf16(a)
    bhi, blo = _split_bf16(b)
    d = lambda x, y: jnp.dot(x, y, preferred_element_type=jnp.float32)
    return d(ahi, bhi) + (d(ahi, blo) + d(alo, bhi))


def _top3_select(d):
    iota0 = jax.lax.broadcasted_iota(
        jnp.int32, (d.shape[0], 1), 0).astype(jnp.float32)
    mins = []
    preds = []
    for k in range(3):
        mval = jnp.min(d, axis=0, keepdims=True)
        idxk = jnp.min(jnp.where(d == mval, iota0, _BIG_F), axis=0,
                       keepdims=True)
        pred = iota0 == idxk
        mins.append(mval)
        preds.append(pred)
        if k < 2:
            d = jnp.where(pred, jnp.inf, d)
    return mins, preds


def _top3_weights(d, m):
    mins, preds = _top3_select(d)
    recips = [1.0 / (dk + _EPS_D) for dk in mins]
    norm = recips[0] + recips[1] + recips[2]
    wt = jnp.zeros(d.shape, jnp.float32)
    for rk, pred in zip(recips, preds):
        wt = jnp.where(pred, rk / norm, wt)
    return wt


def _sqdist(kxyz, uxyz_c, m, n):
    d = jnp.zeros((m, n), jnp.float32)
    for c in range(3):
        diff = kxyz[:, c:c + 1] - uxyz_c[c:c + 1, :]
        d = d + diff * diff
    return d


def _accum_sums(s_ref, y, first):
    part = jnp.concatenate(
        [jnp.sum(y, axis=1, keepdims=True),
         jnp.sum(y * y, axis=1, keepdims=True)], axis=1)

    @pl.when(first)
    def _():
        s_ref[...] = jnp.zeros_like(s_ref)

    s_ref[...] += part


def _sums_of(y):
    return jnp.concatenate(
        [jnp.sum(y, axis=1, keepdims=True),
         jnp.sum(y * y, axis=1, keepdims=True)], axis=1)


def _fp1_chain_body(uxyz_ref, kxyz_ref, w1a_ref, bb_ref, f_ref, w1b_ref,
                    b1_ref, g1_ref, be1_ref, w2_ref, b2_ref, g2_ref, be2_ref,
                    w1a2_ref, ghi_ref, glo_ref, y_scr):
    s1 = jnp.zeros((256, 2), jnp.float32)
    for b in range(B):
        d = _sqdist(kxyz_ref[b], uxyz_ref[b], N2, N1)
        wt = _top3_weights(d, N2)
        hfeat = _dot3(w1a_ref[...], bb_ref[b])
        y = _dot3(hfeat, wt) + _dot3(w1b_ref[...], f_ref[b]) + b1_ref[...]
        y_scr[b] = y
        s1 = s1 + _sums_of(y)
    scale1, shift1 = _bn_scale_shift_v(s1, g1_ref[...], be1_ref[...],
                                       float(B * N1))
    s2 = jnp.zeros((256, 2), jnp.float32)
    for b in range(B):
        a = jnp.maximum(y_scr[b] * scale1 + shift1, 0.0)
        z = _dot3(w2_ref[...], a) + b2_ref[...]
        y_scr[b] = z
        s2 = s2 + _sums_of(z)
    scale2, shift2 = _bn_scale_shift_v(s2, g2_ref[...], be2_ref[...],
                                       float(B * N1))
    for b in range(B):
        a = jnp.maximum(y_scr[b] * scale2 + shift2, 0.0)
        g = _dot3(w1a2_ref[...], a)
        ghi, glo = _split_bf16(g)
        ghi_ref[b] = ghi
        glo_ref[b] = glo


def _bn_scale_shift_v(s, g, be, count):
    mean = s[:, 0:1] / count
    var = s[:, 1:2] / count - mean * mean
    inv = jax.lax.rsqrt(var + _EPS_BN)
    scale = g * inv
    shift = be - mean * scale
    return scale, shift


def _bn_scale_shift(s_ref, g_ref, be_ref, count):
    s = s_ref[...]
    mean = s[:, 0:1] / count
    var = s[:, 1:2] / count - mean * mean
    inv = jax.lax.rsqrt(var + _EPS_BN)
    scale = g_ref[...] * inv
    shift = be_ref[...] - mean * scale
    return scale, shift


def _bn_relu_conv_body(count, x_ref, s_in_ref, g_ref, be_ref, w_ref, b_ref,
                       z_ref, s_out_ref):
    first = pl.program_id(0) == 0
    scale, shift = _bn_scale_shift(s_in_ref, g_ref, be_ref, count)
    a = jnp.maximum(x_ref[0] * scale + shift, 0.0)
    z = _dot3(w_ref[...], a)
    if b_ref is not None:
        z = z + b_ref[...]
    z_ref[0] = z
    if s_out_ref is not None:
        _accum_sums(s_out_ref, z, first)


def _cls_from_z_body(count, z_ref, s4_ref, g2_ref, be2_ref, s5_ref, g3_ref,
                     be3_ref, w_ref, b_ref, out_ref):
    scale2, shift2 = _bn_scale_shift(s4_ref, g2_ref, be2_ref, count)
    scale3, shift3 = _bn_scale_shift(s5_ref, g3_ref, be3_ref, count)
    f = jnp.maximum(z_ref[0] * scale2 + shift2, 0.0)
    a = jnp.maximum(f * scale3 + shift3, 0.0)
    out_ref[0] = _dot3(w_ref[...], a) + b_ref[...]


def _bn_relu_body(count, x_ref, s_in_ref, g_ref, be_ref, f_ref, s_out_ref):
    first = pl.program_id(0) == 0
    scale, shift = _bn_scale_shift(s_in_ref, g_ref, be_ref, count)
    f = jnp.maximum(x_ref[0] * scale + shift, 0.0)
    f_ref[0] = f
    _accum_sums(s_out_ref, f, first)


def _fp2_body(uxyz_ref, kxyz_ref, xf_ref, g1hi_ref, g1lo_ref, w1b_ref, b1_ref,
              y_ref, s_ref):
    b = pl.program_id(0)
    i = pl.program_id(1)
    nt = y_ref.shape[2]
    d = _sqdist(kxyz_ref[0], uxyz_ref[0], N1, nt)
    wt32 = _top3_weights(d, N1)
    wt_hi, wt_lo = _split_bf16(wt32)
    pf32 = jnp.float32
    g1hi = g1hi_ref[0]
    y = (jnp.dot(g1hi, wt_hi, preferred_element_type=pf32)
         + jnp.dot(g1lo_ref[0], wt_hi, preferred_element_type=pf32)
         + jnp.dot(g1hi, wt_lo, preferred_element_type=pf32))
    xf = xf_ref[0]
    w1b = w1b_ref[...]
    for c in range(DIN):
        y = y + w1b[:, c:c + 1] * xf[c:c + 1, :]
    y = y + b1_ref[...]
    y_ref[0] = y
    _accum_sums(s_ref, y, jnp.logical_and(b == 0, i == 0))


def _col(v):
    return v.reshape(-1, 1)


def kernel(input_xyz, sa1_xyz, sa2_xyz, input_features, sa1_features,
           backbone_feat, fp1_w1, fp1_b1, fp1_g1, fp1_be1, fp1_w2, fp1_b2,
           fp1_g2, fp1_be2, fp2_w1, fp2_b1, fp2_g1, fp2_be1, fp2_w2, fp2_b2,
           fp2_g2, fp2_be2, cls_g, cls_be, cls_w, cls_b):
    f32 = jnp.float32
    sa1_xyz_c = sa1_xyz.transpose(0, 2, 1)
    input_xyz_c = input_xyz.transpose(0, 2, 1)
    w1a_fp1 = fp1_w1[:, :256]
    w1b_fp1 = fp1_w1[:, 256:]
    w1a_fp2 = fp2_w1[:, :256]
    w1b_fp2 = fp2_w1[:, 256:]

    full = lambda shp: pl.BlockSpec(shp, lambda b: tuple(0 for _ in shp))
    perb = lambda shp: pl.BlockSpec(
        (1,) + shp, lambda b: (b,) + tuple(0 for _ in shp))

    M1 = float(B * N1)
    M2 = float(B * N)

    whole = lambda shp: pl.BlockSpec(shp, lambda: tuple(0 for _ in shp))
    g1hi, g1lo = pl.pallas_call(
        _fp1_chain_body,
        grid=(),
        in_specs=[whole((B, 3, N1)), whole((B, N2, 3)), whole((256, 256)),
                  whole((B, 256, N2)), whole((B, 128, N1)),
                  whole((256, 128)), whole((256, 1)), whole((256, 1)),
                  whole((256, 1)), whole((256, 256)), whole((256, 1)),
                  whole((256, 1)), whole((256, 1)), whole((256, 256))],
        out_specs=[whole((B, 256, N1)), whole((B, 256, N1))],
        out_shape=[jax.ShapeDtypeStruct((B, 256, N1), jnp.bfloat16),
                   jax.ShapeDtypeStruct((B, 256, N1), jnp.bfloat16)],
        scratch_shapes=[pltpu.VMEM((B, 256, N1), f32)],
    )(sa1_xyz_c, sa2_xyz, w1a_fp1, backbone_feat, sa1_features, w1b_fp1,
      _col(fp1_b1), _col(fp1_g1), _col(fp1_be1), fp1_w2, _col(fp1_b2),
      _col(fp1_g2), _col(fp1_be2), w1a_fp2)

    NT = 1024
    nsteps = N // NT
    y2, s3 = pl.pallas_call(
        _fp2_body,
        grid=(B, nsteps),
        in_specs=[
            pl.BlockSpec((1, 3, NT), lambda b, i: (b, 0, i)),
            pl.BlockSpec((1, N1, 3), lambda b, i: (b, 0, 0)),
            pl.BlockSpec((1, DIN, NT), lambda b, i: (b, 0, i)),
            pl.BlockSpec((1, 256, N1), lambda b, i: (b, 0, 0)),
            pl.BlockSpec((1, 256, N1), lambda b, i: (b, 0, 0)),
            pl.BlockSpec((256, DIN), lambda b, i: (0, 0)),
            pl.BlockSpec((256, 1), lambda b, i: (0, 0)),
        ],
        out_specs=[pl.BlockSpec((1, 256, NT), lambda b, i: (b, 0, i)),
                   pl.BlockSpec((256, 2), lambda b, i: (0, 0))],
        out_shape=[jax.ShapeDtypeStruct((B, 256, N), f32),
                   jax.ShapeDtypeStruct((256, 2), f32)],
    )(input_xyz_c, sa1_xyz, input_features, g1hi, g1lo, w1b_fp2,
      _col(fp2_b1))

    z2, s4 = pl.pallas_call(
        functools.partial(_bn_relu_conv_body, M2),
        grid=(B,),
        in_specs=[perb((256, N)), full((256, 2)), full((256, 1)),
                  full((256, 1)), full((256, 256)), full((256, 1))],
        out_specs=[perb((256, N)), full((256, 2))],
        out_shape=[jax.ShapeDtypeStruct((B, 256, N), f32),
                   jax.ShapeDtypeStruct((256, 2), f32)],
    )(y2, s3, _col(fp2_g1), _col(fp2_be1), fp2_w2, _col(fp2_b2))

    f2, s5 = pl.pallas_call(
        functools.partial(_bn_relu_body, M2),
        grid=(B,),
        in_specs=[perb((256, N)), full((256, 2)), full((256, 1)),
                  full((256, 1))],
        out_specs=[perb((256, N)), full((256, 2))],
        out_shape=[jax.ShapeDtypeStruct((B, 256, N), f32),
                   jax.ShapeDtypeStruct((256, 2), f32)],
    )(z2, s4, _col(fp2_g2), _col(fp2_be2))

    pred = pl.pallas_call(
        functools.partial(_cls_from_z_body, M2),
        grid=(B,),
        in_specs=[perb((256, N)), full((256, 2)), full((256, 1)),
                  full((256, 1)), full((256, 2)), full((256, 1)),
                  full((256, 1)), full((NC, 256)), full((NC, 1))],
        out_specs=perb((NC, N)),
        out_shape=jax.ShapeDtypeStruct((B, NC, N), f32),
    )(z2, s4, _col(fp2_g2), _col(fp2_be2), s5, _col(cls_g), _col(cls_be),
      cls_w, _col(cls_b))

    return (f2, pred)
